# baseline (speedup 1.0000x reference)
import functools

import jax
import jax.numpy as jnp
from jax import lax
from jax.experimental import pallas as pl
from jax.experimental.pallas import tpu as pltpu
from jax.experimental.pallas import tpu_sc as plsc

D = 64
W = 128
NW = 32
NG = W // 16
MAX_NORM = 1.0


def _rsqrt_nr(x):
    i = lax.bitcast_convert_type(x, jnp.int32)
    i = jnp.int32(0x5F3759DF) - lax.shift_right_logical(i, 1)
    y = lax.bitcast_convert_type(i, jnp.float32)
    for _ in range(3):
        y = y * (jnp.float32(1.5) - jnp.float32(0.5) * x * y * y)
    return y


def kernel(xc_padded, table):
    B, S = xc_padded.shape
    n = B * S
    per_w = n // NW
    nwin = per_w // W
    idx = xc_padded.reshape(n // 128, 128)
    mesh = plsc.VectorSubcoreMesh(core_axis_name="core", subcore_axis_name="subcore")
    cp = pltpu.CompilerParams(
        needs_layout_passes=False,
        use_tc_tiling_on_sc=False,
        skip_device_barrier=True,
    )

    @functools.partial(
        pl.kernel,
        out_type=jax.ShapeDtypeStruct((n, D), jnp.float32),
        mesh=mesh,
        compiler_params=cp,
        scratch_types=[
            pltpu.VMEM((per_w // 128, 128), jnp.int32),
            pltpu.VMEM((2, W, D), jnp.float32),
            pltpu.VMEM((NG, 16, 16), jnp.float32),
            pltpu.SemaphoreType.DMA,
            pltpu.SemaphoreType.DMA,
            pltpu.SemaphoreType.DMA,
        ],
    )
    def k(table_hbm, idx_hbm, out_hbm, idx_v, buf, tscr, isem, gsem, osem):
        wid = lax.axis_index("subcore") * 2 + lax.axis_index("core")
        pltpu.async_copy(
            idx_hbm.at[pl.ds(wid * (per_w // 128), per_w // 128)], idx_v, isem
        ).wait()
        lanes = lax.iota(jnp.int32, 16)

        def fire_gathers(win, slot):
            for j in range(W // 16):
                iv = idx_v[win, pl.ds(j * 16, 16)]
                pltpu.async_copy(
                    table_hbm.at[iv], buf.at[slot, pl.ds(j * 16, 16)], gsem
                )

        def drain_gathers(slot):
            pltpu.make_async_copy(
                table_hbm.at[pl.ds(0, W)], buf.at[slot], gsem
            ).wait()

        def compute(slot):
            for g in range(NG):
                for r in range(16):
                    src = buf.at[slot, g * 16 + r]
                    v0 = src[pl.ds(0, 16)]
                    v1 = src[pl.ds(16, 16)]
                    v2 = src[pl.ds(32, 16)]
                    v3 = src[pl.ds(48, 16)]
                    s = (v0 * v0 + v1 * v1) + (v2 * v2 + v3 * v3)
                    cols = jnp.full((16,), r, jnp.int32)
                    plsc.store_scatter(tscr.at[g], [lanes, cols], s)
                sumsq = tscr[g, 0]
                for j in range(1, 16):
                    sumsq = sumsq + tscr[g, j]
                scale16 = jnp.where(
                    sumsq > jnp.float32(MAX_NORM * MAX_NORM),
                    jnp.float32(MAX_NORM) * _rsqrt_nr(sumsq),
                    jnp.float32(1.0),
                )
                for r in range(16):
                    row = buf.at[slot, g * 16 + r]
                    sc = scale16[r]
                    for c4 in range(4):
                        sl = pl.ds(c4 * 16, 16)
                        row[sl] = row[sl] * sc

        def step(w, slot):
            with jax.named_scope("drain_gathers"):
                drain_gathers(slot)

            @pl.when(w + 1 < nwin)
            def _():
                @pl.when(w >= 1)
                def _():
                    with jax.named_scope("wait_write"):
                        pltpu.make_async_copy(
                            buf.at[1 - slot], out_hbm.at[pl.ds(0, W)], osem
                        ).wait()

                with jax.named_scope("fire_gathers"):
                    fire_gathers(w + 1, 1 - slot)

            with jax.named_scope("compute"):
                compute(slot)
            rowbase = (wid * nwin + w) * W
            pltpu.async_copy(buf.at[slot], out_hbm.at[pl.ds(rowbase, W)], osem)

        fire_gathers(0, 0)

        @pl.loop(0, nwin // 2)
        def _(h):
            step(2 * h, 0)
            step(2 * h + 1, 1)

        pltpu.make_async_copy(buf.at[0], out_hbm.at[pl.ds(0, W)], osem).wait()
        pltpu.make_async_copy(buf.at[1], out_hbm.at[pl.ds(0, W)], osem).wait()

    out = k(table, idx)
    return out.reshape(B, S, D)


# speedup vs baseline: 1.1962x; 1.1962x over previous
import functools

import jax
import jax.numpy as jnp
from jax import lax
from jax.experimental import pallas as pl
from jax.experimental.pallas import tpu as pltpu
from jax.experimental.pallas import tpu_sc as plsc

D = 64
NW = 32
MAX_NORM = 1.0


def _rsqrt_nr(x):
    i = lax.bitcast_convert_type(x, jnp.int32)
    i = jnp.int32(0x5F3759DF) - lax.shift_right_logical(i, 1)
    y = lax.bitcast_convert_type(i, jnp.float32)
    for _ in range(3):
        y = y * (jnp.float32(1.5) - jnp.float32(0.5) * x * y * y)
    return y


def kernel(xc_padded, table):
    B, S = xc_padded.shape
    rows_w = B // NW
    nfull = S // 16
    tail = S - nfull * 16
    mesh = plsc.VectorSubcoreMesh(core_axis_name="core", subcore_axis_name="subcore")
    cp = pltpu.CompilerParams(
        needs_layout_passes=False,
        use_tc_tiling_on_sc=False,
        skip_device_barrier=True,
    )
    ns = nfull + (1 if tail else 0)
    s_pad = ns * 16

    @functools.partial(
        pl.kernel,
        out_type=jax.ShapeDtypeStruct((B, S, D), jnp.float32),
        mesh=mesh,
        compiler_params=cp,
        scratch_types=[
            pltpu.VMEM((rows_w, S), jnp.int32),
            pltpu.VMEM((2, s_pad, D), jnp.float32),
            pltpu.VMEM((ns, 16, 16), jnp.float32),
            pltpu.SemaphoreType.DMA,
            pltpu.SemaphoreType.DMA,
            pltpu.SemaphoreType.DMA,
        ],
    )
    def k(table_hbm, idx_hbm, out_hbm, idx_v, buf, tscr, isem, gsem, osem):
        wid = lax.axis_index("subcore") * 2 + lax.axis_index("core")
        pltpu.async_copy(idx_hbm.at[pl.ds(wid * rows_w, rows_w)], idx_v, isem).wait()
        lanes = lax.iota(jnp.int32, 16)

        def fire_gathers(win, slot):
            offs = [j * 16 for j in range(nfull)]
            if tail:
                offs.append(S - 16)
            for o in offs:
                iv = idx_v[win, pl.ds(o, 16)]
                pltpu.async_copy(
                    table_hbm.at[iv], buf.at[slot, pl.ds(o, 16)], gsem
                )

        def drain_gathers(slot):
            pltpu.make_async_copy(
                table_hbm.at[pl.ds(0, s_pad)], buf.at[slot], gsem
            ).wait()

        def group(slot, g, nrows):
            for r in range(nrows):
                src = buf.at[slot, g * 16 + r]
                v0 = src[pl.ds(0, 16)]
                v1 = src[pl.ds(16, 16)]
                v2 = src[pl.ds(32, 16)]
                v3 = src[pl.ds(48, 16)]
                s = (v0 * v0 + v1 * v1) + (v2 * v2 + v3 * v3)
                cols = jnp.full((16,), r, jnp.int32)
                plsc.store_scatter(tscr.at[g], [lanes, cols], s)
            sumsq = tscr[g, 0]
            for j in range(1, 16):
                sumsq = sumsq + tscr[g, j]
            scale16 = jnp.where(
                sumsq > jnp.float32(MAX_NORM * MAX_NORM),
                jnp.float32(MAX_NORM) * _rsqrt_nr(sumsq),
                jnp.float32(1.0),
            )
            for r in range(nrows):
                row = buf.at[slot, g * 16 + r]
                sc = scale16[r]
                for c4 in range(4):
                    sl = pl.ds(c4 * 16, 16)
                    row[sl] = row[sl] * sc

        def compute(slot):
            @pl.loop(0, nfull)
            def _(g):
                group(slot, g, 16)

            if tail:
                group(slot, nfull, tail)

        def step(w, slot):
            with jax.named_scope("drain_gathers"):
                drain_gathers(slot)

            @pl.when(w + 1 < rows_w)
            def _():
                @pl.when(w >= 1)
                def _():
                    with jax.named_scope("wait_write"):
                        pltpu.make_async_copy(
                            buf.at[1 - slot, pl.ds(0, S)], out_hbm.at[0], osem
                        ).wait()

                with jax.named_scope("fire_gathers"):
                    fire_gathers(w + 1, 1 - slot)

            with jax.named_scope("compute"):
                compute(slot)
            pltpu.async_copy(
                buf.at[slot, pl.ds(0, S)], out_hbm.at[wid * rows_w + w], osem
            )

        fire_gathers(0, 0)

        @pl.loop(0, rows_w // 2)
        def _(h):
            step(2 * h, 0)
            step(2 * h + 1, 1)

        pltpu.make_async_copy(buf.at[0, pl.ds(0, S)], out_hbm.at[0], osem).wait()
        pltpu.make_async_copy(buf.at[1, pl.ds(0, S)], out_hbm.at[0], osem).wait()

    return k(table, xc_padded)
